# Initial kernel scaffold; baseline (speedup 1.0000x reference)
import jax
import jax.numpy as jnp
from jax.experimental import pallas as pl


def kernel(feature_indices, feature_values, bias, linear_w, factor_v):
    raise NotImplementedError("write your pallas kernel here")


# speedup vs baseline: 5.3404x; 5.3404x over previous
import functools

import jax
import jax.numpy as jnp
from jax import lax
from jax.experimental import pallas as pl
from jax.experimental.pallas import tpu as pltpu, tpu_sc as plsc

B = 16384
F = 100
FPAD = 128
D = 128
N = 100000

NC = 2
NS = 16
L = 16
NW = NC * NS
ROWS_PER_W = B // NW
GROUPS = ROWS_PER_W // L
FCHUNKS = (24, 24, 24, 28)
GSP = 28
WSP = 32


def _fm_body(fi_hbm, fv_hbm, bias_hbm, lw_hbm, vv_hbm, out_hbm,
             idx_v, x_v, g_v, w_v, bias_v, s_v, out_v, sem):
    wid = lax.axis_index("s") * NC + lax.axis_index("c")
    row_base = wid * ROWS_PER_W

    iota = lax.iota(jnp.int32, L)
    ivg = iota * GSP
    ivw = iota * WSP

    pltpu.sync_copy(bias_hbm, bias_v)
    bias_splat = bias_v[...]

    def group_body(g, carry):
        row0 = row_base + g * L
        pltpu.sync_copy(fi_hbm.at[pl.ds(row0, L)], idx_v)
        pltpu.sync_copy(fv_hbm.at[pl.ds(row0, L)], x_v)

        q = jnp.zeros((L,), jnp.float32)
        lin = jnp.zeros((L,), jnp.float32)

        f0 = 0
        for c, cs in enumerate(FCHUNKS):
            copies = []
            for l in range(L):
                idx_ref = idx_v.at[l, pl.ds(f0, cs)]
                copies.append(pltpu.async_copy(
                    vv_hbm.at[idx_ref],
                    g_v.at[pl.ds(l * GSP, cs), :], sem))
                copies.append(pltpu.async_copy(
                    lw_hbm.at[idx_ref],
                    w_v.at[pl.ds(l * WSP, cs)], sem))
            for cp in copies:
                cp.wait()

            xs = [plsc.load_gather(x_v, [iota, jnp.full((L,), f0 + f, jnp.int32)])
                  for f in range(cs)]

            def d_body(d, q, c=c, cs=cs, xs=xs):
                dsplat = lax.broadcast(d, (L,)).astype(jnp.int32)
                if c == 0:
                    acc = jnp.zeros((L,), jnp.float32)
                else:
                    acc = s_v[d]
                for f in range(cs):
                    gvec = plsc.load_gather(g_v, [ivg + f, dsplat])
                    p = xs[f] * gvec
                    acc = acc + p
                    q = q + p * p
                s_v[d] = acc
                return q

            q = lax.fori_loop(0, D, d_body, q)

            for f in range(cs):
                wvec = plsc.load_gather(w_v, [ivw + f])
                lin = lin + xs[f] * wvec
            f0 += cs

        def inter_body(d, acc):
            sd = s_v[d]
            return acc + sd * sd

        ssq = lax.fori_loop(0, D, inter_body, jnp.zeros((L,), jnp.float32))
        z = bias_splat + lin + 0.5 * (ssq - q)
        out_v[pl.ds(g * L, L)] = 1.0 / (1.0 + jnp.exp(-z))
        return carry

    lax.fori_loop(0, GROUPS, group_body, None)
    pltpu.sync_copy(out_v, out_hbm.at[pl.ds(row_base, ROWS_PER_W)])


_fm = functools.partial(
    pl.kernel,
    out_type=jax.ShapeDtypeStruct((B,), jnp.float32),
    mesh=plsc.VectorSubcoreMesh(core_axis_name="c", subcore_axis_name="s"),
    compiler_params=pltpu.CompilerParams(needs_layout_passes=False),
    scratch_types=[
        pltpu.VMEM((L, FPAD), jnp.int32),
        pltpu.VMEM((L, FPAD), jnp.float32),
        pltpu.VMEM((L * GSP, D), jnp.float32),
        pltpu.VMEM((L * WSP,), jnp.float32),
        pltpu.VMEM((L,), jnp.float32),
        pltpu.VMEM((D, L), jnp.float32),
        pltpu.VMEM((ROWS_PER_W,), jnp.float32),
        pltpu.SemaphoreType.DMA,
    ],
)(_fm_body)


def kernel(feature_indices, feature_values, bias, linear_w, factor_v):
    fi = feature_indices.astype(jnp.int32)
    fv = feature_values.astype(jnp.float32)
    fi = jnp.pad(fi, ((0, 0), (0, FPAD - F)))
    fv = jnp.pad(fv, ((0, 0), (0, FPAD - F)))
    bias16 = jnp.broadcast_to(bias.reshape(()), (L,))
    return _fm(fi, fv, bias16, linear_w.reshape(N), factor_v)


# speedup vs baseline: 21.3660x; 4.0008x over previous
import functools

import jax
import jax.numpy as jnp
from jax import lax
from jax.experimental import pallas as pl
from jax.experimental.pallas import tpu as pltpu, tpu_sc as plsc

B = 16384
F = 100
FPAD = 128
D = 128
N = 100000

NC = 2
NS = 16
L = 16
NW = NC * NS
ROWS_PER_W = B // NW
GROUPS = ROWS_PER_W // L
FCHUNKS = (24, 24, 24, 28)
GSP = 28
WSP = 32
DC = D // L


def _fm_body(fi_hbm, fv_hbm, bias_hbm, lw_hbm, vv_hbm, out_hbm,
             idx_v, x_v, g_v, w_v, bias_v, s_v, q_v, out_v, sem):
    wid = lax.axis_index("s") * NC + lax.axis_index("c")
    row_base = wid * ROWS_PER_W

    iota = lax.iota(jnp.int32, L)
    ivw = iota * WSP

    pltpu.sync_copy(bias_hbm, bias_v)
    bias_vec = bias_v[...]

    def group_body(g, carry):
        row0 = row_base + g * L
        pltpu.sync_copy(fi_hbm.at[pl.ds(row0, L)], idx_v)
        pltpu.sync_copy(fv_hbm.at[pl.ds(row0, L)], x_v)

        lin = jnp.zeros((L,), jnp.float32)

        f0 = 0
        for c, cs in enumerate(FCHUNKS):
            copies = []
            for l in range(L):
                idx_ref = idx_v.at[l, pl.ds(f0, cs)]
                copies.append(pltpu.async_copy(
                    vv_hbm.at[idx_ref],
                    g_v.at[pl.ds(l * GSP, cs), :], sem))
                copies.append(pltpu.async_copy(
                    lw_hbm.at[idx_ref],
                    w_v.at[pl.ds(l * WSP, cs)], sem))
            for cp in copies:
                cp.wait()

            def row_body(r, carry2, c=c, cs=cs, f0=f0):
                if c == 0:
                    s = [jnp.zeros((L,), jnp.float32) for _ in range(DC)]
                    q = jnp.zeros((L,), jnp.float32)
                else:
                    s = [s_v[r, pl.ds(cc * L, L)] for cc in range(DC)]
                    q = q_v[r]
                xblk = {k: x_v[r, pl.ds(k * L, L)]
                        for k in range(f0 // L, (f0 + cs - 1) // L + 1)}
                for j in range(cs):
                    xb = lax.broadcast(xblk[(f0 + j) // L][(f0 + j) % L], (L,))
                    gr = r * GSP + j
                    for cc in range(DC):
                        gv = g_v[gr, pl.ds(cc * L, L)]
                        p = xb * gv
                        s[cc] = s[cc] + p
                        q = q + p * p
                for cc in range(DC):
                    s_v[r, pl.ds(cc * L, L)] = s[cc]
                q_v[r] = q
                return carry2

            lax.fori_loop(0, L, row_body, None)

            xs = [plsc.load_gather(x_v, [iota, jnp.full((L,), f0 + j, jnp.int32)])
                  for j in range(cs)]
            for j in range(cs):
                wvec = plsc.load_gather(w_v, [ivw + j])
                lin = lin + xs[j] * wvec
            f0 += cs

        def epi_body(r, acc):
            sr = [s_v[r, pl.ds(cc * L, L)] for cc in range(DC)]
            ssqv = sr[0] * sr[0]
            for cc in range(1, DC):
                ssqv = ssqv + sr[cc] * sr[cc]
            inter = 0.5 * (jnp.sum(ssqv) - jnp.sum(q_v[r]))
            return jnp.where(iota == r, lax.broadcast(inter, (L,)), acc)

        inter_vec = lax.fori_loop(0, L, epi_body, jnp.zeros((L,), jnp.float32))
        z = bias_vec + lin + inter_vec
        out_v[pl.ds(g * L, L)] = 1.0 / (1.0 + jnp.exp(-z))
        return carry

    lax.fori_loop(0, GROUPS, group_body, None)
    pltpu.sync_copy(out_v, out_hbm.at[pl.ds(row_base, ROWS_PER_W)])


_fm = functools.partial(
    pl.kernel,
    out_type=jax.ShapeDtypeStruct((B,), jnp.float32),
    mesh=plsc.VectorSubcoreMesh(core_axis_name="c", subcore_axis_name="s"),
    compiler_params=pltpu.CompilerParams(needs_layout_passes=False),
    scratch_types=[
        pltpu.VMEM((L, FPAD), jnp.int32),
        pltpu.VMEM((L, FPAD), jnp.float32),
        pltpu.VMEM((L * GSP, D), jnp.float32),
        pltpu.VMEM((L * WSP,), jnp.float32),
        pltpu.VMEM((L,), jnp.float32),
        pltpu.VMEM((L, D), jnp.float32),
        pltpu.VMEM((L, L), jnp.float32),
        pltpu.VMEM((ROWS_PER_W,), jnp.float32),
        pltpu.SemaphoreType.DMA,
    ],
)(_fm_body)


def kernel(feature_indices, feature_values, bias, linear_w, factor_v):
    fi = feature_indices.astype(jnp.int32)
    fv = feature_values.astype(jnp.float32)
    fi = jnp.pad(fi, ((0, 0), (0, FPAD - F)))
    fv = jnp.pad(fv, ((0, 0), (0, FPAD - F)))
    bias16 = jnp.broadcast_to(bias.reshape(()), (L,))
    return _fm(fi, fv, bias16, linear_w.reshape(N), factor_v)


# speedup vs baseline: 27.4470x; 1.2846x over previous
import functools

import jax
import jax.numpy as jnp
from jax import lax
from jax.experimental import pallas as pl
from jax.experimental.pallas import tpu as pltpu, tpu_sc as plsc

B = 16384
F = 100
FPAD = 128
D = 128
N = 100000

NC = 2
NS = 16
L = 16
NW = NC * NS
ROWS_PER_W = B // NW
GROUPS = ROWS_PER_W // L
FCHUNKS = (16, 16, 16, 16, 16, 20)
GSP = 20
WSP = 32
DC = D // L


def _fm_body(fi_hbm, fv_hbm, bias_hbm, lw_hbm, vv_hbm, out_hbm,
             idx_v, x_v, g_v0, g_v1, w_v0, w_v1, bias_v, s_v, q_v, out_v,
             sem0, sem1):
    wid = lax.axis_index("s") * NC + lax.axis_index("c")
    row_base = wid * ROWS_PER_W

    iota = lax.iota(jnp.int32, L)
    ivw = iota * WSP

    gbuf = (g_v0, g_v1)
    wbuf = (w_v0, w_v1)
    sems = (sem0, sem1)
    starts = []
    acc = 0
    for cs in FCHUNKS:
        starts.append(acc)
        acc += cs

    pltpu.sync_copy(bias_hbm, bias_v)
    bias_vec = bias_v[...]

    def fire(c):
        f0, cs = starts[c], FCHUNKS[c]
        g_v, w_v, sem = gbuf[c % 2], wbuf[c % 2], sems[c % 2]
        copies = []
        for l in range(L):
            idx_ref = idx_v.at[l, pl.ds(f0, cs)]
            copies.append(pltpu.async_copy(
                vv_hbm.at[idx_ref],
                g_v.at[pl.ds(l * GSP, cs), :], sem))
            copies.append(pltpu.async_copy(
                lw_hbm.at[idx_ref],
                w_v.at[pl.ds(l * WSP, cs)], sem))
        return copies

    def group_body(g, carry):
        row0 = row_base + g * L
        pltpu.sync_copy(fi_hbm.at[pl.ds(row0, L)], idx_v)
        pltpu.sync_copy(fv_hbm.at[pl.ds(row0, L)], x_v)

        lin = jnp.zeros((L,), jnp.float32)

        inflight = fire(0)
        for c, cs in enumerate(FCHUNKS):
            f0 = starts[c]
            g_v, w_v = gbuf[c % 2], wbuf[c % 2]
            nxt = fire(c + 1) if c + 1 < len(FCHUNKS) else []
            for cp in inflight:
                cp.wait()
            inflight = nxt

            def row_body(r, carry2, c=c, cs=cs, f0=f0):
                if c == 0:
                    s = [jnp.zeros((L,), jnp.float32) for _ in range(DC)]
                    q = jnp.zeros((L,), jnp.float32)
                else:
                    s = [s_v[r, pl.ds(cc * L, L)] for cc in range(DC)]
                    q = q_v[r]
                xblk = {k: x_v[r, pl.ds(k * L, L)]
                        for k in range(f0 // L, (f0 + cs - 1) // L + 1)}
                for j in range(cs):
                    xb = lax.broadcast(xblk[(f0 + j) // L][(f0 + j) % L], (L,))
                    gr = r * GSP + j
                    for cc in range(DC):
                        gv = g_v[gr, pl.ds(cc * L, L)]
                        p = xb * gv
                        s[cc] = s[cc] + p
                        q = q + p * p
                for cc in range(DC):
                    s_v[r, pl.ds(cc * L, L)] = s[cc]
                q_v[r] = q
                return carry2

            lax.fori_loop(0, L, row_body, None)

            for j in range(cs):
                xsj = plsc.load_gather(
                    x_v, [iota, jnp.full((L,), f0 + j, jnp.int32)])
                wvec = plsc.load_gather(w_v, [ivw + j])
                lin = lin + xsj * wvec

        def epi_body(r, acc):
            sr = [s_v[r, pl.ds(cc * L, L)] for cc in range(DC)]
            ssqv = sr[0] * sr[0]
            for cc in range(1, DC):
                ssqv = ssqv + sr[cc] * sr[cc]
            inter = 0.5 * (jnp.sum(ssqv) - jnp.sum(q_v[r]))
            return jnp.where(iota == r, lax.broadcast(inter, (L,)), acc)

        inter_vec = lax.fori_loop(0, L, epi_body, jnp.zeros((L,), jnp.float32))
        z = bias_vec + lin + inter_vec
        out_v[pl.ds(g * L, L)] = 1.0 / (1.0 + jnp.exp(-z))
        return carry

    lax.fori_loop(0, GROUPS, group_body, None)
    pltpu.sync_copy(out_v, out_hbm.at[pl.ds(row_base, ROWS_PER_W)])


_fm = functools.partial(
    pl.kernel,
    out_type=jax.ShapeDtypeStruct((B,), jnp.float32),
    mesh=plsc.VectorSubcoreMesh(core_axis_name="c", subcore_axis_name="s"),
    compiler_params=pltpu.CompilerParams(needs_layout_passes=False),
    scratch_types=[
        pltpu.VMEM((L, FPAD), jnp.int32),
        pltpu.VMEM((L, FPAD), jnp.float32),
        pltpu.VMEM((L * GSP, D), jnp.float32),
        pltpu.VMEM((L * GSP, D), jnp.float32),
        pltpu.VMEM((L * WSP,), jnp.float32),
        pltpu.VMEM((L * WSP,), jnp.float32),
        pltpu.VMEM((L,), jnp.float32),
        pltpu.VMEM((L, D), jnp.float32),
        pltpu.VMEM((L, L), jnp.float32),
        pltpu.VMEM((ROWS_PER_W,), jnp.float32),
        pltpu.SemaphoreType.DMA,
        pltpu.SemaphoreType.DMA,
    ],
)(_fm_body)


def kernel(feature_indices, feature_values, bias, linear_w, factor_v):
    fi = feature_indices.astype(jnp.int32)
    fv = feature_values.astype(jnp.float32)
    fi = jnp.pad(fi, ((0, 0), (0, FPAD - F)))
    fv = jnp.pad(fv, ((0, 0), (0, FPAD - F)))
    bias16 = jnp.broadcast_to(bias.reshape(()), (L,))
    return _fm(fi, fv, bias16, linear_w.reshape(N), factor_v)
